# baseline (speedup 1.0000x reference)
import jax
import jax.numpy as jnp
from jax.experimental import pallas as pl
from jax.experimental.pallas import tpu as pltpu

_N = 10000
_F = 128
_FUSSY = 4
_BM = 400
_NB = _N // _BM
_R = 3


def _body(mu1_ref, sig1_ref, mu2_ref, sig2_ref, x_ref, adj_ref, w1_ref,
          w2_ref, b1_ref, b2_ref, out_ref, ring_ref, y_ref, y2_ref, sems):
    l = pl.program_id(0)
    i = pl.program_id(1)
    j = l * _NB + i

    def fetch(jj, slot):
        bi = jax.lax.rem(jj, _NB)
        return pltpu.make_async_copy(
            adj_ref.at[pl.ds(bi * _BM, _BM), :], ring_ref.at[slot],
            sems.at[slot])

    @pl.when(j == 0)
    def _prologue():
        for s in range(_R):
            fetch(s, s).start()
        y_ref[...] = jnp.dot(x_ref[...].astype(jnp.bfloat16),
                             w1_ref[...].astype(jnp.bfloat16),
                             preferred_element_type=jnp.float32
                             ).astype(jnp.bfloat16)

    @pl.when(jnp.logical_and(l == 1, i == 0))
    def _swap_to_y2():
        y_ref[...] = y2_ref[...]

    slot = jax.lax.rem(j, _R)
    fetch(j, slot).wait()

    z = jnp.dot(ring_ref[slot].astype(jnp.bfloat16), y_ref[...],
                preferred_element_type=jnp.float32)
    z = z + jnp.where(l == 0, b1_ref[...], b2_ref[...])
    acc = None
    for k in range(_FUSSY):
        m = jnp.where(l == 0, mu1_ref[k], mu2_ref[k])
        s = jnp.where(l == 0, sig1_ref[k], sig2_ref[k])
        d = z - m
        t = jnp.exp(d * d * (-1.0 / (s * s)))
        acc = t if acc is None else acc + t
    gated = z * (acc * (1.0 / _FUSSY))

    @pl.when(l == 0)
    def _store_layer1():
        y2_ref[pl.ds(i * _BM, _BM), :] = jnp.dot(
            gated.astype(jnp.bfloat16), w2_ref[...].astype(jnp.bfloat16),
            preferred_element_type=jnp.float32).astype(jnp.bfloat16)

    @pl.when(l == 1)
    def _store_layer2():
        out_ref[...] = gated

    nj = j + _R
    @pl.when(nj < 2 * _NB)
    def _refill():
        fetch(nj, slot).start()


def kernel(x, adj, W1, b1, mu1, sig1, W2, b2, mu2, sig2):
    return pl.pallas_call(
        _body,
        grid=(2, _NB),
        in_specs=[
            pl.BlockSpec(memory_space=pltpu.SMEM),
            pl.BlockSpec(memory_space=pltpu.SMEM),
            pl.BlockSpec(memory_space=pltpu.SMEM),
            pl.BlockSpec(memory_space=pltpu.SMEM),
            pl.BlockSpec((_N, _F), lambda l, i: (0, 0)),
            pl.BlockSpec(memory_space=pl.ANY),
            pl.BlockSpec((_F, _F), lambda l, i: (0, 0)),
            pl.BlockSpec((_F, _F), lambda l, i: (0, 0)),
            pl.BlockSpec((1, _F), lambda l, i: (0, 0)),
            pl.BlockSpec((1, _F), lambda l, i: (0, 0)),
        ],
        out_specs=pl.BlockSpec((_BM, _F), lambda l, i: (i * l, 0)),
        out_shape=jax.ShapeDtypeStruct((_N, _F), jnp.float32),
        scratch_shapes=[
            pltpu.VMEM((_R, _BM, _N), jnp.float32),
            pltpu.VMEM((_N, _F), jnp.bfloat16),
            pltpu.VMEM((_N, _F), jnp.bfloat16),
            pltpu.SemaphoreType.DMA((_R,)),
        ],
        compiler_params=pltpu.CompilerParams(
            vmem_limit_bytes=100 * 1024 * 1024,
        ),
    )(mu1, sig1, mu2, sig2, x, adj, W1, W2,
      b1.reshape(1, _F), b2.reshape(1, _F))


# speedup vs baseline: 1.0193x; 1.0193x over previous
import jax
import jax.numpy as jnp
from jax.experimental import pallas as pl
from jax.experimental.pallas import tpu as pltpu

_N = 10000
_F = 128
_FUSSY = 4
_BM = 400


def _body(mu1_ref, sig1_ref, mu2_ref, sig2_ref, x_ref, adj_ref, w1_ref,
          w2_ref, b1_ref, b2_ref, out_ref, y_ref, y2_ref):
    l = pl.program_id(0)
    i = pl.program_id(1)

    @pl.when(jnp.logical_and(l == 0, i == 0))
    def _init_y1():
        y_ref[...] = jnp.dot(x_ref[...], w1_ref[...],
                             preferred_element_type=jnp.float32)

    @pl.when(jnp.logical_and(l == 1, i == 0))
    def _swap_to_y2():
        y_ref[...] = y2_ref[...]

    z = jnp.dot(adj_ref[...], y_ref[...],
                preferred_element_type=jnp.float32)
    z = z + jnp.where(l == 0, b1_ref[...], b2_ref[...])
    acc = None
    for k in range(_FUSSY):
        m = jnp.where(l == 0, mu1_ref[k], mu2_ref[k])
        s = jnp.where(l == 0, sig1_ref[k], sig2_ref[k])
        d = z - m
        t = jnp.exp(d * d * (-1.0 / (s * s)))
        acc = t if acc is None else acc + t
    gated = z * (acc * (1.0 / _FUSSY))

    @pl.when(l == 0)
    def _store_layer1():
        y2_ref[pl.ds(i * _BM, _BM), :] = jnp.dot(
            gated, w2_ref[...], preferred_element_type=jnp.float32)

    @pl.when(l == 1)
    def _store_layer2():
        out_ref[...] = gated


def kernel(x, adj, W1, b1, mu1, sig1, W2, b2, mu2, sig2):
    return pl.pallas_call(
        _body,
        grid=(2, _N // _BM),
        in_specs=[
            pl.BlockSpec(memory_space=pltpu.SMEM),
            pl.BlockSpec(memory_space=pltpu.SMEM),
            pl.BlockSpec(memory_space=pltpu.SMEM),
            pl.BlockSpec(memory_space=pltpu.SMEM),
            pl.BlockSpec((_N, _F), lambda l, i: (0, 0)),
            pl.BlockSpec((_BM, _N), lambda l, i: (i, 0)),
            pl.BlockSpec((_F, _F), lambda l, i: (0, 0)),
            pl.BlockSpec((_F, _F), lambda l, i: (0, 0)),
            pl.BlockSpec((1, _F), lambda l, i: (0, 0)),
            pl.BlockSpec((1, _F), lambda l, i: (0, 0)),
        ],
        out_specs=pl.BlockSpec((_BM, _F), lambda l, i: (i * l, 0)),
        out_shape=jax.ShapeDtypeStruct((_N, _F), jnp.float32),
        scratch_shapes=[
            pltpu.VMEM((_N, _F), jnp.float32),
            pltpu.VMEM((_N, _F), jnp.float32),
        ],
        compiler_params=pltpu.CompilerParams(
            vmem_limit_bytes=100 * 1024 * 1024,
        ),
    )(mu1, sig1, mu2, sig2, x, adj, W1, W2,
      b1.reshape(1, _F), b2.reshape(1, _F))
